# Initial kernel scaffold; baseline (speedup 1.0000x reference)
import jax
import jax.numpy as jnp
from jax.experimental import pallas as pl


def kernel(node_x, net_x, edge_weight_sink_to_net, params, edge_index_sink_to_net, edge_index_source_to_net):
    raise NotImplementedError("write your pallas kernel here")


# speedup vs baseline: 1.4635x; 1.4635x over previous
import functools

import numpy as np
import jax
import jax.numpy as jnp
from jax import lax
from jax.experimental import pallas as pl
from jax.experimental.pallas import tpu as pltpu
from jax.experimental.pallas import tpu_sc as plsc

_NUM_LAYER = 3
_EMB = 512
_N_NODE = 10000
_N_NET = 10000
_E = 160000
_DROP_P = 0.4
_NCHUNK = 4
_CW = 128
_B = 128
_NW = 32
_ACC_ROWS = 10112
_RPT = _ACC_ROWS // 16
_BM = 1000

_U32 = np.uint64(0xFFFFFFFF)


def _threefry2x32(k0, k1, x0, x1):
    x0 = np.asarray(x0, np.uint64)
    x1 = np.asarray(x1, np.uint64)
    ks = [np.uint64(k0), np.uint64(k1),
          np.uint64(k0) ^ np.uint64(k1) ^ np.uint64(0x1BD11BDA)]
    rot = ((13, 15, 26, 6), (17, 29, 16, 24))
    x0 = (x0 + ks[0]) & _U32
    x1 = (x1 + ks[1]) & _U32
    for i in range(5):
        for r in rot[i % 2]:
            x0 = (x0 + x1) & _U32
            r64 = np.uint64(r)
            x1 = ((x1 << r64 | x1 >> (np.uint64(32) - r64)) & _U32) ^ x0
        x0 = (x0 + ks[(i + 1) % 3]) & _U32
        x1 = (x1 + ks[(i + 2) % 3] + np.uint64(i + 1)) & _U32
    return x0.astype(np.uint32), x1.astype(np.uint32)


def _uniform01(seed, n):
    idx = np.arange(n, dtype=np.uint64)
    hi = (idx >> np.uint64(32)).astype(np.uint32)
    lo = (idx & _U32).astype(np.uint32)
    a, b = _threefry2x32(0, seed, hi, lo)
    bits = a ^ b
    u = ((bits >> np.uint32(9)) | np.uint32(0x3F800000)).view(np.float32)
    return np.maximum(np.float32(0.0), u - np.float32(1.0))


_MASK = _uniform01(42, _E) >= _DROP_P
_KEEP = np.nonzero(_MASK)[0].astype(np.int32)
_KN = int(_KEEP.shape[0])


def _round_up(n, m):
    return (n + m - 1) // m * m


_UPAD = _round_up(_E, _B * _NW)
_WPAD = _round_up(_KN, _B * _NW)


def _lrelu(x):
    return jnp.where(x >= 0, x, 0.1 * x)


def _lin_body(x_ref, w_ref, b_ref, o_ref):
    y = jnp.dot(x_ref[...], w_ref[...], preferred_element_type=jnp.float32)
    o_ref[...] = _lrelu(y + b_ref[...])


def _linear_lrelu(x, w, b):
    m, k = x.shape
    n = w.shape[1]
    return pl.pallas_call(
        _lin_body,
        grid=(m // _BM,),
        in_specs=[
            pl.BlockSpec((_BM, k), lambda i: (i, 0)),
            pl.BlockSpec((k, n), lambda i: (0, 0)),
            pl.BlockSpec((1, n), lambda i: (0, 0)),
        ],
        out_specs=pl.BlockSpec((_BM, n), lambda i: (i, 0)),
        out_shape=jax.ShapeDtypeStruct((m, n), jnp.float32),
    )(x, w, b.reshape(1, n))


def _msg_body(x_ref, w_ref, b_ref, o_ref):
    y = _lrelu(jnp.dot(x_ref[...], w_ref[...], preferred_element_type=jnp.float32)
               + b_ref[...])
    for c in range(_NCHUNK):
        o_ref[c] = y[:, c * _CW:(c + 1) * _CW]


def _msg_mm(x, w, b):
    m, k = x.shape
    n = w.shape[1]
    return pl.pallas_call(
        _msg_body,
        grid=(m // _BM,),
        in_specs=[
            pl.BlockSpec((_BM, k), lambda i: (i, 0)),
            pl.BlockSpec((k, n), lambda i: (0, 0)),
            pl.BlockSpec((1, n), lambda i: (0, 0)),
        ],
        out_specs=pl.BlockSpec((_NCHUNK, _BM, _CW), lambda i: (0, i, 0)),
        out_shape=jax.ShapeDtypeStruct((_NCHUNK, m, _CW), jnp.float32),
    )(x, w, b.reshape(1, n))


def _hn_body(hn_ref, agg_ref, wt_ref, wb_ref, b_ref, opre_ref, ores_ref):
    hn = hn_ref[...]
    acc = jnp.dot(hn, wt_ref[...], preferred_element_type=jnp.float32)
    wb = wb_ref[...]
    for core in range(2):
        for c in range(_NCHUNK):
            acc += jnp.dot(agg_ref[core, c], wb[c * _CW:(c + 1) * _CW],
                           preferred_element_type=jnp.float32)
    pre = _lrelu(acc + b_ref[...])
    for c in range(_NCHUNK):
        opre_ref[c] = pre[:, c * _CW:(c + 1) * _CW]
    ores_ref[...] = pre + hn


def _hn_mm(hn, agg, wt, wb, b):
    m, n = hn.shape
    return pl.pallas_call(
        _hn_body,
        grid=(m // _BM,),
        in_specs=[
            pl.BlockSpec((_BM, n), lambda i: (i, 0)),
            pl.BlockSpec((2, _NCHUNK, _BM, _CW), lambda i: (0, 0, i, 0)),
            pl.BlockSpec((n, n), lambda i: (0, 0)),
            pl.BlockSpec((n, n), lambda i: (0, 0)),
            pl.BlockSpec((1, n), lambda i: (0, 0)),
        ],
        out_specs=[
            pl.BlockSpec((_NCHUNK, _BM, _CW), lambda i: (0, i, 0)),
            pl.BlockSpec((_BM, n), lambda i: (i, 0)),
        ],
        out_shape=[
            jax.ShapeDtypeStruct((_NCHUNK, m, _CW), jnp.float32),
            jax.ShapeDtypeStruct((m, n), jnp.float32),
        ],
    )(hn, agg, wt, wb, b.reshape(1, n))


def _h_body(h_ref, agg_ref, wt_ref, wb_ref, b_ref, o_ref):
    h = h_ref[...]
    acc = jnp.dot(h, wt_ref[...], preferred_element_type=jnp.float32)
    wb = wb_ref[...]
    for core in range(2):
        for c in range(_NCHUNK):
            acc += jnp.dot(agg_ref[core, c], wb[c * _CW:(c + 1) * _CW],
                           preferred_element_type=jnp.float32)
    o_ref[...] = _lrelu(acc + b_ref[...]) + h


def _h_mm(h, agg, wt, wb, b):
    m, n = h.shape
    return pl.pallas_call(
        _h_body,
        grid=(m // _BM,),
        in_specs=[
            pl.BlockSpec((_BM, n), lambda i: (i, 0)),
            pl.BlockSpec((2, _NCHUNK, _BM, _CW), lambda i: (0, 0, i, 0)),
            pl.BlockSpec((n, n), lambda i: (0, 0)),
            pl.BlockSpec((n, n), lambda i: (0, 0)),
            pl.BlockSpec((1, n), lambda i: (0, 0)),
        ],
        out_specs=pl.BlockSpec((_BM, n), lambda i: (i, 0)),
        out_shape=jax.ShapeDtypeStruct((m, n), jnp.float32),
    )(h, agg, wt, wb, b.reshape(1, n))


def _head_node_body(h0, h1, h2, h3, w1, b1, w2, b2, wf, bf, o_ref):
    hs = (h0, h1, h2, h3)
    acc = b1[...].astype(jnp.float32) * jnp.ones((_BM, 1), jnp.float32)
    for i in range(4):
        acc += jnp.dot(hs[i][...], w1[i], preferred_element_type=jnp.float32)
    t = _lrelu(acc)
    t = _lrelu(jnp.dot(t, w2[...], preferred_element_type=jnp.float32) + b2[...])
    o_ref[...] = jnp.dot(t, wf[...], preferred_element_type=jnp.float32) + bf[...]


def _head_node(h_list, w1, b1, w2, b2, wf, bf):
    m, n = h_list[0].shape
    w1r = w1.reshape(4, n, 256)
    return pl.pallas_call(
        _head_node_body,
        grid=(m // _BM,),
        in_specs=[pl.BlockSpec((_BM, n), lambda i: (i, 0)) for _ in range(4)] + [
            pl.BlockSpec((4, n, 256), lambda i: (0, 0, 0)),
            pl.BlockSpec((1, 256), lambda i: (0, 0)),
            pl.BlockSpec((256, 256), lambda i: (0, 0)),
            pl.BlockSpec((1, 256), lambda i: (0, 0)),
            pl.BlockSpec((256, 1), lambda i: (0, 0)),
            pl.BlockSpec((1, 1), lambda i: (0, 0)),
        ],
        out_specs=pl.BlockSpec((_BM, 1), lambda i: (i, 0)),
        out_shape=jax.ShapeDtypeStruct((m, 1), jnp.float32),
    )(*h_list, w1r, b1.reshape(1, 256), w2, b2.reshape(1, 256), wf,
      bf.reshape(1, 1))


def _head_net_body(h0, h1, h2, h3, w1, b1, w2, b2, o_ref):
    hs = (h0, h1, h2, h3)
    acc = b1[...].astype(jnp.float32) * jnp.ones((_BM, 1), jnp.float32)
    for i in range(4):
        acc += jnp.dot(hs[i][...], w1[i], preferred_element_type=jnp.float32)
    t = _lrelu(acc)
    o_ref[...] = jnp.abs(
        _lrelu(jnp.dot(t, w2[...], preferred_element_type=jnp.float32) + b2[...]))


def _head_net(h_list, w1, b1, w2, b2):
    m, n = h_list[0].shape
    w1r = w1.reshape(4, n, 64)
    return pl.pallas_call(
        _head_net_body,
        grid=(m // _BM,),
        in_specs=[pl.BlockSpec((_BM, n), lambda i: (i, 0)) for _ in range(4)] + [
            pl.BlockSpec((4, n, 64), lambda i: (0, 0, 0)),
            pl.BlockSpec((1, 64), lambda i: (0, 0)),
            pl.BlockSpec((64, 64), lambda i: (0, 0)),
            pl.BlockSpec((1, 64), lambda i: (0, 0)),
        ],
        out_specs=pl.BlockSpec((_BM, 64), lambda i: (i, 0)),
        out_shape=jax.ShapeDtypeStruct((m, 64), jnp.float32),
    )(*h_list, w1r, b1.reshape(1, 64), w2, b2.reshape(1, 64))


@functools.lru_cache(maxsize=None)
def _sc_agg(nu, nw):
    upt = nu // _NW
    wpt = nw // _NW
    mesh = plsc.VectorSubcoreMesh(core_axis_name="c", subcore_axis_name="s",
                                  num_cores=2, num_subcores=16)

    def body(*refs):
        if wpt:
            (gu_ref, du_ref, gw_ref, dw_ref, w_ref, tab_ref, z_ref, out_ref,
             rows_v, gi_v, di_v, w_v, accum, sem) = refs
        else:
            (gu_ref, du_ref, tab_ref, z_ref, out_ref,
             rows_v, gi_v, di_v, w_v, accum, sem) = refs
        c = lax.axis_index("c")
        s = lax.axis_index("s")
        wid = c * 16 + s
        for cf in range(_NCHUNK):
            pltpu.sync_copy(z_ref, accum.at[pl.ds(s * _RPT, _RPT)])
            plsc.subcore_barrier()

            def ubody(b, carry):
                off = wid * upt + b * _B
                pltpu.sync_copy(gu_ref.at[cf].at[pl.ds(off, _B)], gi_v)
                pltpu.sync_copy(du_ref.at[pl.ds(off, _B)], di_v)
                pltpu.async_copy(tab_ref.at[gi_v], rows_v, sem).wait()
                pltpu.sync_copy(rows_v, accum.at[di_v], add=True)
                return carry

            lax.fori_loop(0, upt // _B, ubody, 0)

            if wpt:
                def wbody(b, carry):
                    off = wid * wpt + b * _B
                    pltpu.sync_copy(gw_ref.at[cf].at[pl.ds(off, _B)], gi_v)
                    pltpu.sync_copy(dw_ref.at[pl.ds(off, _B)], di_v)
                    pltpu.sync_copy(w_ref.at[pl.ds(off, _B)], w_v)
                    pltpu.async_copy(tab_ref.at[gi_v], rows_v, sem).wait()

                    def ebody(e, cc):
                        ev = jnp.zeros((16,), jnp.int32) + e
                        wb = plsc.load_gather(w_v, [ev])
                        for kk in range(_CW // 16):
                            rows_v[e, pl.ds(kk * 16, 16)] = (
                                rows_v[e, pl.ds(kk * 16, 16)] * wb)
                        return cc

                    lax.fori_loop(0, _B, ebody, 0)
                    pltpu.sync_copy(rows_v, accum.at[di_v], add=True)
                    return carry

                lax.fori_loop(0, wpt // _B, wbody, 0)

            plsc.subcore_barrier()
            pltpu.sync_copy(accum.at[pl.ds(s * _RPT, _RPT)],
                            out_ref.at[c, cf].at[pl.ds(s * _RPT, _RPT)])
            plsc.subcore_barrier()

    return pl.kernel(
        body,
        out_type=jax.ShapeDtypeStruct((2, _NCHUNK, _ACC_ROWS, _CW), jnp.float32),
        mesh=mesh,
        compiler_params=pltpu.CompilerParams(needs_layout_passes=False),
        scratch_types=[
            pltpu.VMEM((_B, _CW), jnp.float32),
            pltpu.VMEM((_B,), jnp.int32),
            pltpu.VMEM((_B,), jnp.int32),
            pltpu.VMEM((_B,), jnp.float32),
            pltpu.VMEM_SHARED((_ACC_ROWS, _CW), jnp.float32),
            pltpu.SemaphoreType.DMA,
        ],
    )


def _pad1(x, n, val):
    return jnp.concatenate(
        [x, jnp.full((n - x.shape[0],), val, x.dtype)])


def _chunk4(g):
    off = jnp.asarray(
        (np.arange(_NCHUNK, dtype=np.int32) * _N_NODE)[:, None])
    return g[None, :] + off


def kernel(node_x, net_x, edge_weight_sink_to_net, params,
           edge_index_sink_to_net, edge_index_source_to_net):
    p = params
    ew = edge_weight_sink_to_net

    h = _linear_lrelu(node_x, p['enc_W1'], p['enc_b1'])
    h = _linear_lrelu(h, p['enc_W2'], p['enc_b2'])
    hn = _linear_lrelu(net_x, p['net_W'], p['net_b'])

    src_nodes = edge_index_source_to_net[0].astype(jnp.int32)
    src_nets = edge_index_source_to_net[1].astype(jnp.int32)
    sink_nodes = edge_index_sink_to_net[0][_KEEP].astype(jnp.int32)
    sink_nets = edge_index_sink_to_net[1][_KEEP].astype(jnp.int32)
    kw = ew[_KEEP]

    gu4 = _chunk4(_pad1(src_nodes, _UPAD, 0))
    du = _pad1(src_nets, _UPAD, _N_NET)
    gw4 = _chunk4(_pad1(sink_nodes, _WPAD, 0))
    dw = _pad1(sink_nets, _WPAD, _N_NET)
    wv = _pad1(kw, _WPAD, 0.0)
    gn4 = _chunk4(_pad1(sink_nets, _WPAD, 0))
    dn = _pad1(sink_nodes, _WPAD, _N_NET)
    zeros = jnp.zeros((_RPT, _CW), jnp.float32)

    net_agg_fn = _sc_agg(_UPAD, _WPAD)
    node_agg_fn = _sc_agg(_WPAD, 0)

    h_list = [h]
    hn_list = [hn]
    for l in range(_NUM_LAYER):
        lp = p['layers'][l]
        msg_cm = _msg_mm(h_list[l], lp['W_msg'], lp['b_msg'])
        nagg = net_agg_fn(gu4, du, gw4, dw, wv,
                          msg_cm.reshape(_NCHUNK * _N_NODE, _CW), zeros)
        hn_pre_cm, hn_res = _hn_mm(hn_list[l], nagg, lp['W_net'][:_EMB],
                                   lp['W_net'][_EMB:], lp['b_net'])
        dagg = node_agg_fn(gn4, dn,
                           hn_pre_cm.reshape(_NCHUNK * _N_NET, _CW), zeros)
        h_new = _h_mm(h_list[l], dagg, lp['W_node'][:_EMB],
                      lp['W_node'][_EMB:], lp['b_node'])
        h_list.append(h_new)
        hn_list.append(hn_res)

    node_out = _head_node(h_list, p['fc1n_W'], p['fc1n_b'],
                          p['fc2n_W'], p['fc2n_b'], p['final_W'], p['final_b'])
    net_out = _head_net(hn_list, p['fc1e_W'], p['fc1e_b'],
                        p['fc2e_W'], p['fc2e_b'])
    return node_out, net_out
